# Initial kernel scaffold; baseline (speedup 1.0000x reference)
import jax
import jax.numpy as jnp
from jax.experimental import pallas as pl


def kernel(T, table):
    raise NotImplementedError("write your pallas kernel here")


# speedup vs baseline: 42.2956x; 42.2956x over previous
import functools

import jax
import jax.numpy as jnp
from jax import lax
from jax.experimental import pallas as pl
from jax.experimental.pallas import tpu as pltpu
from jax.experimental.pallas import tpu_sc as plsc

_MAXD = 128
_H = 16
_T = 2048
_WLEN = 4112
_NW = 32
_ROWS_PER_W = (_H * _T) // _NW
_FIRE = 8
_TTL = 304
_BAND0 = 1896
_FILL_LO = _BAND0 + 8
_FILL_HI = _BAND0 + 296


def _sc_bias(tt_all):
    mesh = plsc.VectorSubcoreMesh(core_axis_name="c", subcore_axis_name="s")

    @functools.partial(
        pl.kernel,
        out_type=jax.ShapeDtypeStruct((_H * _T * _T,), jnp.float32),
        mesh=mesh,
        scratch_types=[
            pltpu.VMEM((8 * _WLEN,), jnp.float32),
            pltpu.SemaphoreType.DMA,
        ],
    )
    def k(tt_hbm, out_hbm, w_v, sem):
        cid = lax.axis_index("c")
        sid = lax.axis_index("s")
        wid = sid * 2 + cid
        head = wid // 2
        half = wid % 2

        for r in range(8):
            base = r * _WLEN
            src_off = pl.multiple_of((r * _H + head) * _TTL, 8)
            pltpu.sync_copy(
                tt_hbm.at[pl.ds(src_off, _TTL)],
                w_v.at[pl.ds(base + _BAND0, _TTL)],
            )
            c_lo = w_v[pl.ds(base + _BAND0, 16)]
            c_hi = w_v[pl.ds(base + _BAND0 + 288, 16)]

            def fill(g, carry, base=base, c_lo=c_lo, c_hi=c_hi):
                off = pl.multiple_of(g * 16, 16)
                w_v[pl.ds(base + off, 16)] = c_lo
                w_v[pl.ds(base + _FILL_HI + off, 16)] = c_hi
                return carry

            lax.fori_loop(0, _FILL_LO // 16, fill, 0)

        row0 = head * _T + half * _ROWS_PER_W

        def emit(it, carry):
            base8 = _T - half * _ROWS_PER_W - it * 8
            copies = []
            for b in range(_FIRE):
                r = (8 - b) % 8
                q8 = base8 - (8 if b else 0)
                src = w_v.at[pl.ds(pl.multiple_of(r * _WLEN + q8, 8), _T)]
                d_off = pl.multiple_of((row0 + it * _FIRE + b) * _T, _T)
                dst = out_hbm.at[pl.ds(d_off, _T)]
                copies.append(pltpu.async_copy(src, dst, sem))
            for c in copies:
                c.wait()
            return carry

        lax.fori_loop(0, _ROWS_PER_W // _FIRE, emit, 0)

    return k(tt_all)


def kernel(T, table):
    j = jnp.arange(_TTL)
    r = jnp.arange(8)
    rows = jnp.clip(j[None, :] + r[:, None] - 24, 0, 2 * _MAXD)
    tt_all = jnp.transpose(table[rows], (0, 2, 1)).reshape(-1)
    out2d = _sc_bias(tt_all)
    return out2d.reshape(1, _H, _T, _T)


# speedup vs baseline: 42.5378x; 1.0057x over previous
import functools

import jax
import jax.numpy as jnp
from jax import lax
from jax.experimental import pallas as pl
from jax.experimental.pallas import tpu as pltpu
from jax.experimental.pallas import tpu_sc as plsc

_MAXD = 128
_H = 16
_T = 2048
_WLEN = 4112
_NW = 32
_ROWS_PER_W = (_H * _T) // _NW
_FIRE = 32
_TTL = 304
_BAND0 = 1896
_FILL_LO = _BAND0 + 8
_FILL_HI = _BAND0 + 296


def _sc_bias(tt_all):
    mesh = plsc.VectorSubcoreMesh(core_axis_name="c", subcore_axis_name="s")

    @functools.partial(
        pl.kernel,
        out_type=jax.ShapeDtypeStruct((_H * _T * _T,), jnp.float32),
        mesh=mesh,
        scratch_types=[
            pltpu.VMEM((8 * _WLEN,), jnp.float32),
            pltpu.SemaphoreType.DMA,
        ],
    )
    def k(tt_hbm, out_hbm, w_v, sem):
        cid = lax.axis_index("c")
        sid = lax.axis_index("s")
        wid = sid * 2 + cid
        head = wid // 2
        half = wid % 2

        for r in range(8):
            base = r * _WLEN
            src_off = pl.multiple_of((r * _H + head) * _TTL, 8)
            pltpu.sync_copy(
                tt_hbm.at[pl.ds(src_off, _TTL)],
                w_v.at[pl.ds(base + _BAND0, _TTL)],
            )
            c_lo = w_v[pl.ds(base + _BAND0, 16)]
            c_hi = w_v[pl.ds(base + _BAND0 + 288, 16)]

            def fill(g, carry, base=base, c_lo=c_lo, c_hi=c_hi):
                off = pl.multiple_of(g * 16, 16)
                w_v[pl.ds(base + off, 16)] = c_lo
                w_v[pl.ds(base + _FILL_HI + off, 16)] = c_hi
                return carry

            lax.fori_loop(0, _FILL_LO // 16, fill, 0)

        row0 = head * _T + half * _ROWS_PER_W

        def emit(it, carry):
            base8 = _T - half * _ROWS_PER_W - it * _FIRE
            copies = []
            for b in range(_FIRE):
                u = b % 8
                r = (8 - u) % 8
                q8 = base8 - (b - u) - (8 if u else 0)
                src = w_v.at[pl.ds(pl.multiple_of(r * _WLEN + q8, 8), _T)]
                d_off = pl.multiple_of((row0 + it * _FIRE + b) * _T, _T)
                dst = out_hbm.at[pl.ds(d_off, _T)]
                copies.append(pltpu.async_copy(src, dst, sem))
            for c in copies:
                c.wait()
            return carry

        lax.fori_loop(0, _ROWS_PER_W // _FIRE, emit, 0)

    return k(tt_all)


def kernel(T, table):
    j = jnp.arange(_TTL)
    r = jnp.arange(8)
    rows = jnp.clip(j[None, :] + r[:, None] - 24, 0, 2 * _MAXD)
    tt_all = jnp.transpose(table[rows], (0, 2, 1)).reshape(-1)
    out2d = _sc_bias(tt_all)
    return out2d.reshape(1, _H, _T, _T)
